# baseline (speedup 1.0000x reference)
import functools

import jax
import jax.numpy as jnp
from jax import lax
from jax.experimental import pallas as pl
from jax.experimental.pallas import tpu as pltpu

B, D, E, K = 4096, 1024, 8, 2
BM = 512


def _moe_block(x32_ref, xbf_ref, w_ref, b_ref, gw_ref, gb_ref, out_ref):
    e = pl.program_id(1)

    x32 = x32_ref[...]
    gw = gw_ref[...]
    logits = lax.dot_general(
        x32, gw, (((1,), (1,)), ((), ())),
        preferred_element_type=jnp.float32) + gb_ref[...]
    cols = lax.broadcasted_iota(jnp.int32, logits.shape, 1)
    idx1 = jnp.argmax(logits, axis=1, keepdims=True)
    v1 = jnp.max(logits, axis=1, keepdims=True)
    l2 = jnp.where(cols == idx1, -jnp.inf, logits)
    idx2 = jnp.argmax(l2, axis=1, keepdims=True)
    v2 = jnp.max(l2, axis=1, keepdims=True)
    w1 = 1.0 / (1.0 + jnp.exp(v2 - v1))
    w2 = 1.0 - w1
    w_col = jnp.where(idx1 == e, w1, jnp.where(idx2 == e, w2, 0.0))

    acc = lax.dot_general(
        xbf_ref[...], w_ref[0],
        (((1,), (1,)), ((), ())),
        preferred_element_type=jnp.float32)
    acc = acc + b_ref[0]
    contrib = w_col * acc

    @pl.when(e == 0)
    def _init():
        out_ref[...] = contrib

    @pl.when(e != 0)
    def _acc():
        out_ref[...] += contrib


@functools.partial(jax.jit)
def _moe(x, W, b, gate_W, gate_b):
    xbf = x.astype(jnp.bfloat16)
    Wbf = W.astype(jnp.bfloat16)
    grid = (B // BM, E)
    return pl.pallas_call(
        _moe_block,
        grid=grid,
        in_specs=[
            pl.BlockSpec((BM, D), lambda g, e: (g, 0)),
            pl.BlockSpec((BM, D), lambda g, e: (g, 0)),
            pl.BlockSpec((1, D, D), lambda g, e: (e, 0, 0)),
            pl.BlockSpec((1, 1, D), lambda g, e: (e, 0, 0)),
            pl.BlockSpec((E, D), lambda g, e: (0, 0)),
            pl.BlockSpec((1, E), lambda g, e: (0, 0)),
        ],
        out_specs=pl.BlockSpec((BM, D), lambda g, e: (g, 0)),
        out_shape=jax.ShapeDtypeStruct((B, D), jnp.float32),
        compiler_params=pltpu.CompilerParams(
            dimension_semantics=("parallel", "arbitrary"),
        ),
    )(x, xbf, Wbf, b.reshape(E, 1, D), gate_W, gate_b.reshape(1, E))


def kernel(x, W, b, gate_W, gate_b):
    return _moe(x, W, b, gate_W, gate_b)


# speedup vs baseline: 1.4920x; 1.4920x over previous
import functools

import jax
import jax.numpy as jnp
from jax import lax
from jax.experimental import pallas as pl
from jax.experimental.pallas import tpu as pltpu

B, D, E, K = 4096, 1024, 8, 2
BM = 512


def _moe_block(x_ref, w_ref, b_ref, gw_ref, gb_ref, out_ref):
    x32 = x_ref[...]

    logits = lax.dot_general(
        x32, gw_ref[...], (((1,), (1,)), ((), ())),
        preferred_element_type=jnp.float32) + gb_ref[...]
    cols = lax.broadcasted_iota(jnp.int32, logits.shape, 1)
    idx1 = jnp.argmax(logits, axis=1, keepdims=True)
    v1 = jnp.max(logits, axis=1, keepdims=True)
    l2 = jnp.where(cols == idx1, -jnp.inf, logits)
    idx2 = jnp.argmax(l2, axis=1, keepdims=True)
    v2 = jnp.max(l2, axis=1, keepdims=True)
    w1 = 1.0 / (1.0 + jnp.exp(v2 - v1))
    w_mat = jnp.where(cols == idx1, w1,
                      jnp.where(cols == idx2, 1.0 - w1, 0.0))

    acc = lax.dot_general(
        w_mat, b_ref[...], (((1,), (0,)), ((), ())),
        preferred_element_type=jnp.float32)

    xbf = x32.astype(jnp.bfloat16)
    for e in range(E):
        y = lax.dot_general(
            xbf, w_ref[e], (((1,), (1,)), ((), ())),
            preferred_element_type=jnp.float32)
        acc = acc + w_mat[:, e:e + 1] * y

    out_ref[...] = acc


@functools.partial(jax.jit)
def _moe(x, W, b, gate_W, gate_b):
    Wbf = W.astype(jnp.bfloat16)
    return pl.pallas_call(
        _moe_block,
        grid=(B // BM,),
        in_specs=[
            pl.BlockSpec((BM, D), lambda g: (g, 0)),
            pl.BlockSpec((E, D, D), lambda g: (0, 0, 0)),
            pl.BlockSpec((E, D), lambda g: (0, 0)),
            pl.BlockSpec((E, D), lambda g: (0, 0)),
            pl.BlockSpec((1, E), lambda g: (0, 0)),
        ],
        out_specs=pl.BlockSpec((BM, D), lambda g: (g, 0)),
        out_shape=jax.ShapeDtypeStruct((B, D), jnp.float32),
        compiler_params=pltpu.CompilerParams(
            dimension_semantics=("arbitrary",),
        ),
    )(x, Wbf, b, gate_W, gate_b.reshape(1, E))


def kernel(x, W, b, gate_W, gate_b):
    return _moe(x, W, b, gate_W, gate_b)


# speedup vs baseline: 1.6420x; 1.1005x over previous
import functools

import jax
import jax.numpy as jnp
from jax import lax
from jax.experimental import pallas as pl
from jax.experimental.pallas import tpu as pltpu

B, D, E, K = 4096, 1024, 8, 2
BM = 512
G = B // BM


def _gate_and_bias(x32, gw_ref, gb_ref, b_ref):
    logits = lax.dot_general(
        x32, gw_ref[...], (((1,), (1,)), ((), ())),
        preferred_element_type=jnp.float32) + gb_ref[...]
    cols = lax.broadcasted_iota(jnp.int32, logits.shape, 1)
    idx1 = jnp.argmax(logits, axis=1, keepdims=True)
    v1 = jnp.max(logits, axis=1, keepdims=True)
    l2 = jnp.where(cols == idx1, -jnp.inf, logits)
    idx2 = jnp.argmax(l2, axis=1, keepdims=True)
    v2 = jnp.max(l2, axis=1, keepdims=True)
    w1 = 1.0 / (1.0 + jnp.exp(v2 - v1))
    w_mat = jnp.where(cols == idx1, w1,
                      jnp.where(cols == idx2, 1.0 - w1, 0.0))
    acc = lax.dot_general(
        w_mat, b_ref[...], (((1,), (0,)), ((), ())),
        preferred_element_type=jnp.float32)
    return w_mat, acc


def _moe_block(x_ref, w_hbm, b_ref, gw_ref, gb_ref, out_ref,
               wbf_ref, wtmp_ref, sems):
    g = pl.program_id(0)
    x32 = x_ref[...]
    w_mat, acc0 = _gate_and_bias(x32, gw_ref, gb_ref, b_ref)
    xbf = x32.astype(jnp.bfloat16)

    def expert_dot(acc, e, wv):
        y = lax.dot_general(
            xbf, wv, (((1,), (1,)), ((), ())),
            preferred_element_type=jnp.float32)
        return acc + w_mat[:, e:e + 1] * y

    @pl.when(g == 0)
    def _first_block():
        pltpu.make_async_copy(w_hbm.at[0], wtmp_ref.at[0], sems.at[0]).start()
        acc = acc0
        for e in range(E):
            if e + 1 < E:
                s = (e + 1) % 2
                pltpu.make_async_copy(
                    w_hbm.at[e + 1], wtmp_ref.at[s], sems.at[s]).start()
            pltpu.make_async_copy(
                w_hbm.at[e], wtmp_ref.at[e % 2], sems.at[e % 2]).wait()
            wv = wtmp_ref[e % 2].astype(jnp.bfloat16)
            wbf_ref[e] = wv
            acc = expert_dot(acc, e, wv)
        out_ref[...] = acc

    @pl.when(g != 0)
    def _rest():
        acc = acc0
        for e in range(E):
            acc = expert_dot(acc, e, wbf_ref[e])
        out_ref[...] = acc


@functools.partial(jax.jit)
def _moe(x, W, b, gate_W, gate_b):
    return pl.pallas_call(
        _moe_block,
        grid=(G,),
        in_specs=[
            pl.BlockSpec((BM, D), lambda g: (g, 0)),
            pl.BlockSpec(memory_space=pl.ANY),
            pl.BlockSpec((E, D), lambda g: (0, 0)),
            pl.BlockSpec((E, D), lambda g: (0, 0)),
            pl.BlockSpec((1, E), lambda g: (0, 0)),
        ],
        out_specs=pl.BlockSpec((BM, D), lambda g: (g, 0)),
        out_shape=jax.ShapeDtypeStruct((B, D), jnp.float32),
        scratch_shapes=[
            pltpu.VMEM((E, D, D), jnp.bfloat16),
            pltpu.VMEM((2, D, D), jnp.float32),
            pltpu.SemaphoreType.DMA((2,)),
        ],
        compiler_params=pltpu.CompilerParams(
            dimension_semantics=("arbitrary",),
        ),
    )(x, W, b, gate_W, gate_b.reshape(1, E))


def kernel(x, W, b, gate_W, gate_b):
    return _moe(x, W, b, gate_W, gate_b)
